# Initial kernel scaffold; baseline (speedup 1.0000x reference)
import jax
import jax.numpy as jnp
from jax.experimental import pallas as pl


def kernel(x, lengths, emb, W, b):
    raise NotImplementedError("write your pallas kernel here")


# speedup vs baseline: 2.2133x; 2.2133x over previous
import functools

import jax
import jax.numpy as jnp
from jax import lax
from jax.experimental import pallas as pl
from jax.experimental.pallas import tpu as pltpu
from jax.experimental.pallas import tpu_sc as plsc

B = 16384
H = 200
D = 64
NC = 2
NS = 16
NW = NC * NS
RW = B // NW
CH = 128
NCH = RW // CH


def _sc_body(xt_hbm, emb_hbm, out_hbm, idx_v, rows_v, acc_v, sem):
    c = lax.axis_index("c")
    s = lax.axis_index("s")
    wid = c * NS + s
    base = wid * NCH

    def load_idx(l):
        pltpu.sync_copy(xt_hbm.at[l, pl.ds(base, NCH)], idx_v)

    def gather(dst_ref):
        cps = [
            pltpu.async_copy(
                emb_hbm.at[idx_v.at[j]], dst_ref.at[pl.ds(j * CH, CH)], sem
            )
            for j in range(NCH)
        ]
        for cp in cps:
            cp.wait()

    load_idx(0)
    gather(acc_v)

    def step(l, carry):
        load_idx(l)
        gather(rows_v)

        @plsc.parallel_loop(0, RW, unroll=8)
        def _accum(r):
            for k in range(D // 16):
                sl = pl.ds(k * 16, 16)
                plsc.addupdate(acc_v.at[r, sl], rows_v[r, sl])

        return carry

    lax.fori_loop(1, H, step, 0)

    pltpu.sync_copy(acc_v, out_hbm.at[pl.ds(wid * RW, RW)])


@jax.jit
def _sc_sum(xt, emb):
    mesh = plsc.VectorSubcoreMesh(core_axis_name="c", subcore_axis_name="s")
    fn = pl.kernel(
        _sc_body,
        out_type=jax.ShapeDtypeStruct((B, D), jnp.float32),
        mesh=mesh,
        scratch_types=[
            pltpu.VMEM((NCH, CH), jnp.int32),
            pltpu.VMEM((RW, D), jnp.float32),
            pltpu.VMEM((RW, D), jnp.float32),
            pltpu.SemaphoreType.DMA,
        ],
        compiler_params=pltpu.CompilerParams(use_tc_tiling_on_sc=False),
    )
    return fn(xt, emb)


BLK = 512


def _tc_body(x_ref, sum_ref, w_ref, b_ref, o_ref):
    cnt = jnp.sum((x_ref[...] != 0).astype(jnp.float32), axis=1, keepdims=True)
    mean = sum_ref[...] / (cnt + 1e-6)
    o_ref[...] = (
        lax.dot_general(
            mean, w_ref[...], (((1,), (1,)), ((), ())),
            preferred_element_type=jnp.float32,
        )
        + b_ref[...]
    )


@jax.jit
def _tc_finish(x, summed, W, b2):
    return pl.pallas_call(
        _tc_body,
        grid=(B // BLK,),
        in_specs=[
            pl.BlockSpec((BLK, H), lambda i: (i, 0)),
            pl.BlockSpec((BLK, D), lambda i: (i, 0)),
            pl.BlockSpec((D, D), lambda i: (0, 0)),
            pl.BlockSpec((1, D), lambda i: (0, 0)),
        ],
        out_specs=pl.BlockSpec((BLK, D), lambda i: (i, 0)),
        out_shape=jax.ShapeDtypeStruct((B, D), jnp.float32),
    )(x, summed, W, b2)


def kernel(x, lengths, emb, W, b):
    x = jnp.asarray(x, jnp.int32)
    xt = x.T.reshape(H, B // CH, CH)
    summed = _sc_sum(xt, emb)
    return _tc_finish(x, summed, W, b.reshape(1, D))


# speedup vs baseline: 3.3249x; 1.5022x over previous
import functools

import jax
import jax.numpy as jnp
from jax import lax
from jax.experimental import pallas as pl
from jax.experimental.pallas import tpu as pltpu
from jax.experimental.pallas import tpu_sc as plsc

B = 16384
H = 200
D = 64
NC = 2
NS = 16
NW = NC * NS
RW = B // NW
CH = 128
NCH = RW // CH


def _sc_body(
    xt_hbm, emb_hbm, out_hbm,
    idx_a, idx_b, rows_a, rows_b, acc_v, sem_a, sem_b, isem,
):
    c = lax.axis_index("c")
    s = lax.axis_index("s")
    wid = c * NS + s
    base = wid * NCH

    def idx_src(l):
        return xt_hbm.at[l, pl.ds(base, NCH)]

    def fire_idx(l, idx_ref):
        pltpu.async_copy(idx_src(l), idx_ref, isem)

    def wait_idx(idx_ref):
        pltpu.make_async_copy(idx_src(0), idx_ref, isem).wait()

    def fire_gathers(idx_ref, rows_ref, sem):
        for j in range(NCH):
            pltpu.async_copy(
                emb_hbm.at[idx_ref.at[j]], rows_ref.at[pl.ds(j * CH, CH)], sem
            )

    def wait_gathers(rows_ref, sem):
        pltpu.make_async_copy(emb_hbm.at[pl.ds(0, RW)], rows_ref, sem).wait()

    def accumulate(rows_ref):
        @plsc.parallel_loop(0, RW, unroll=8)
        def _acc(r):
            for k in range(D // 16):
                sl = pl.ds(k * 16, 16)
                plsc.addupdate(acc_v.at[r, sl], rows_ref[r, sl])

    @plsc.parallel_loop(0, RW, unroll=8)
    def _zero(r):
        for k in range(D // 16):
            acc_v[r, pl.ds(k * 16, 16)] = jnp.zeros((16,), jnp.float32)

    pltpu.sync_copy(idx_src(0), idx_a)
    fire_gathers(idx_a, rows_a, sem_a)
    fire_idx(1, idx_b)

    def pair(i, carry):
        a = 2 * i
        wait_idx(idx_b)
        fire_gathers(idx_b, rows_b, sem_b)
        wait_gathers(rows_a, sem_a)
        fire_idx(a + 2, idx_a)
        accumulate(rows_a)

        wait_idx(idx_a)
        fire_gathers(idx_a, rows_a, sem_a)
        wait_gathers(rows_b, sem_b)
        fire_idx(a + 3, idx_b)
        accumulate(rows_b)
        return carry

    lax.fori_loop(0, (H - 2) // 2, pair, 0)

    wait_idx(idx_b)
    fire_gathers(idx_b, rows_b, sem_b)
    wait_gathers(rows_a, sem_a)
    accumulate(rows_a)
    wait_gathers(rows_b, sem_b)
    accumulate(rows_b)

    pltpu.sync_copy(acc_v, out_hbm.at[pl.ds(wid * RW, RW)])


@jax.jit
def _sc_sum(xt, emb):
    mesh = plsc.VectorSubcoreMesh(core_axis_name="c", subcore_axis_name="s")
    fn = pl.kernel(
        _sc_body,
        out_type=jax.ShapeDtypeStruct((B, D), jnp.float32),
        mesh=mesh,
        scratch_types=[
            pltpu.VMEM((NCH, CH), jnp.int32),
            pltpu.VMEM((NCH, CH), jnp.int32),
            pltpu.VMEM((RW, D), jnp.float32),
            pltpu.VMEM((RW, D), jnp.float32),
            pltpu.VMEM((RW, D), jnp.float32),
            pltpu.SemaphoreType.DMA,
            pltpu.SemaphoreType.DMA,
            pltpu.SemaphoreType.DMA,
        ],
        compiler_params=pltpu.CompilerParams(use_tc_tiling_on_sc=False),
    )
    return fn(xt, emb)


BLK = 512


def _tc_body(x_ref, sum_ref, w_ref, b_ref, o_ref):
    cnt = jnp.sum((x_ref[...] != 0).astype(jnp.float32), axis=1, keepdims=True)
    mean = sum_ref[...] / (cnt + 1e-6)
    o_ref[...] = (
        lax.dot_general(
            mean, w_ref[...], (((1,), (1,)), ((), ())),
            preferred_element_type=jnp.float32,
        )
        + b_ref[...]
    )


@jax.jit
def _tc_finish(x, summed, W, b2):
    return pl.pallas_call(
        _tc_body,
        grid=(B // BLK,),
        in_specs=[
            pl.BlockSpec((BLK, H), lambda i: (i, 0)),
            pl.BlockSpec((BLK, D), lambda i: (i, 0)),
            pl.BlockSpec((D, D), lambda i: (0, 0)),
            pl.BlockSpec((1, D), lambda i: (0, 0)),
        ],
        out_specs=pl.BlockSpec((BLK, D), lambda i: (i, 0)),
        out_shape=jax.ShapeDtypeStruct((B, D), jnp.float32),
    )(x, summed, W, b2)


def kernel(x, lengths, emb, W, b):
    x = jnp.asarray(x, jnp.int32)
    xt = x.T.reshape(H, B // CH, CH)
    summed = _sc_sum(xt, emb)
    return _tc_finish(x, summed, W, b.reshape(1, D))
